# baseline (device time: 4253350 ns/iter reference)
import jax
import jax.numpy as jnp
from jax import lax
from jax.experimental import pallas as pl
from jax.experimental.pallas import tpu as pltpu

N_DEV_Z = 2
NCHUNK = 8


def kernel(x):
    m_per, n = x.shape

    def body(x_ref, out_ref, local_sem, send_sem, recv_sem):
        my_x = lax.axis_index("x")
        my_y = lax.axis_index("y")
        my_z = lax.axis_index("z")
        peer = (my_x, my_y, 1 - my_z)

        barrier_sem = pltpu.get_barrier_semaphore()
        pl.semaphore_signal(
            barrier_sem, inc=1, device_id=peer,
            device_id_type=pl.DeviceIdType.MESH,
        )
        pl.semaphore_wait(barrier_sem, 1)

        local = pltpu.make_async_copy(
            x_ref, out_ref.at[pl.ds(my_z * m_per, m_per), :], local_sem
        )
        local.start()

        rows = m_per // NCHUNK
        rdmas = []
        for k in range(NCHUNK):
            rdma = pltpu.make_async_remote_copy(
                src_ref=x_ref.at[pl.ds(k * rows, rows), :],
                dst_ref=out_ref.at[pl.ds(my_z * m_per + k * rows, rows), :],
                send_sem=send_sem.at[k],
                recv_sem=recv_sem.at[k],
                device_id=peer,
                device_id_type=pl.DeviceIdType.MESH,
            )
            rdma.start()
            rdmas.append(rdma)

        local.wait()
        for rdma in rdmas:
            rdma.wait()

    out_shape = jax.ShapeDtypeStruct((N_DEV_Z * m_per, n), x.dtype)
    return pl.pallas_call(
        body,
        out_shape=out_shape,
        in_specs=[pl.BlockSpec(memory_space=pl.ANY)],
        out_specs=pl.BlockSpec(memory_space=pl.ANY),
        scratch_shapes=[
            pltpu.SemaphoreType.DMA,
            pltpu.SemaphoreType.DMA((NCHUNK,)),
            pltpu.SemaphoreType.DMA((NCHUNK,)),
        ],
        compiler_params=pltpu.CompilerParams(collective_id=0),
    )(x)


# device time: 1613717 ns/iter; 2.6357x vs baseline; 2.6357x over previous
import jax
import jax.numpy as jnp
from jax import lax
from jax.experimental import pallas as pl
from jax.experimental.pallas import tpu as pltpu

N_DEV_Z = 2
LCHUNK = 16


def kernel(x):
    m_per, n = x.shape
    rows = m_per // LCHUNK

    def body(x_ref, out_ref, vmem_buf, in_sems, out_sems, send_sem, recv_sem):
        my_x = lax.axis_index("x")
        my_y = lax.axis_index("y")
        my_z = lax.axis_index("z")
        peer = (my_x, my_y, 1 - my_z)
        base = my_z * m_per

        barrier_sem = pltpu.get_barrier_semaphore()
        pl.semaphore_signal(
            barrier_sem, inc=1, device_id=peer,
            device_id_type=pl.DeviceIdType.MESH,
        )
        pl.semaphore_wait(barrier_sem, 1)

        rdma = pltpu.make_async_remote_copy(
            src_ref=x_ref,
            dst_ref=out_ref.at[pl.ds(base, m_per), :],
            send_sem=send_sem,
            recv_sem=recv_sem,
            device_id=peer,
            device_id_type=pl.DeviceIdType.MESH,
        )
        rdma.start()

        outs = []
        for k in range(LCHUNK):
            slot = k % 2
            if k >= 2:
                outs[k - 2].wait()
            cin = pltpu.make_async_copy(
                x_ref.at[pl.ds(k * rows, rows), :],
                vmem_buf.at[slot],
                in_sems.at[slot],
            )
            cin.start()
            cin.wait()
            cout = pltpu.make_async_copy(
                vmem_buf.at[slot],
                out_ref.at[pl.ds(base + k * rows, rows), :],
                out_sems.at[slot],
            )
            cout.start()
            outs.append(cout)
        outs[-2].wait()
        outs[-1].wait()

        rdma.wait()

    out_shape = jax.ShapeDtypeStruct((N_DEV_Z * m_per, n), x.dtype)
    return pl.pallas_call(
        body,
        out_shape=out_shape,
        in_specs=[pl.BlockSpec(memory_space=pl.ANY)],
        out_specs=pl.BlockSpec(memory_space=pl.ANY),
        scratch_shapes=[
            pltpu.VMEM((2, rows, n), x.dtype),
            pltpu.SemaphoreType.DMA((2,)),
            pltpu.SemaphoreType.DMA((2,)),
            pltpu.SemaphoreType.DMA,
            pltpu.SemaphoreType.DMA,
        ],
        compiler_params=pltpu.CompilerParams(collective_id=0),
    )(x)


# device time: 827667 ns/iter; 5.1390x vs baseline; 1.9497x over previous
import jax
import jax.numpy as jnp
from jax import lax
from jax.experimental import pallas as pl
from jax.experimental.pallas import tpu as pltpu

MESH = pl.DeviceIdType.MESH
C = 8
LCHUNK = 16


def kernel(x):
    m_per, n = x.shape
    nq = m_per // 4
    sub = nq // C
    h = C // 2
    lrows = m_per // LCHUNK

    def body(
        x_ref, out_ref, vmem_buf, lin_sems, lout_sems,
        zs, zr, h1sn, h1sp, h1rp, h1rn, h2sn, h2sp, h2rp, h2rn,
    ):
        my_x = lax.axis_index("x")
        my_y = lax.axis_index("y")
        my_z = lax.axis_index("z")
        zpeer = (my_x, my_y, 1 - my_z)

        eq = my_x == my_y
        nxt = (jnp.where(eq, my_x, 1 - my_x), jnp.where(eq, 1 - my_y, my_y), my_z)
        prv = (jnp.where(eq, 1 - my_x, my_x), jnp.where(eq, my_y, 1 - my_y), my_z)
        r = 2 * my_x + jnp.where(my_x == 0, my_y, 1 - my_y)
        rp1 = (r + 1) % 4
        rm1 = (r + 3) % 4
        rp2 = (r + 2) % 4

        mine_base = my_z * m_per
        foreign_base = (1 - my_z) * m_per

        barrier_sem = pltpu.get_barrier_semaphore()
        for nbr in (zpeer, nxt, prv):
            pl.semaphore_signal(barrier_sem, inc=1, device_id=nbr,
                                device_id_type=MESH)
        pl.semaphore_wait(barrier_sem, 3)

        z_rdmas = []
        for c in range(C):
            off = r * nq + c * sub
            rd = pltpu.make_async_remote_copy(
                src_ref=x_ref.at[pl.ds(off, sub), :],
                dst_ref=out_ref.at[pl.ds(mine_base + off, sub), :],
                send_sem=zs.at[c], recv_sem=zr.at[c],
                device_id=zpeer, device_id_type=MESH,
            )
            rd.start()
            z_rdmas.append(rd)

        louts = []
        for k in range(LCHUNK):
            slot = k % 2
            if k >= 2:
                louts[k - 2].wait()
            cin = pltpu.make_async_copy(
                x_ref.at[pl.ds(k * lrows, lrows), :],
                vmem_buf.at[slot], lin_sems.at[slot],
            )
            cin.start()
            cin.wait()
            cout = pltpu.make_async_copy(
                vmem_buf.at[slot],
                out_ref.at[pl.ds(mine_base + k * lrows, lrows), :],
                lout_sems.at[slot],
            )
            cout.start()
            louts.append(cout)

        h1_rdmas = []
        for c in range(C):
            z_rdmas[c].wait_recv()
            off = foreign_base + r * nq + c * sub
            src = out_ref.at[pl.ds(off, sub), :]
            rdn = pltpu.make_async_remote_copy(
                src_ref=src, dst_ref=out_ref.at[pl.ds(off, sub), :],
                send_sem=h1sn.at[c], recv_sem=h1rp.at[c],
                device_id=nxt, device_id_type=MESH,
            )
            rdn.start()
            rdp = pltpu.make_async_remote_copy(
                src_ref=src, dst_ref=out_ref.at[pl.ds(off, sub), :],
                send_sem=h1sp.at[c], recv_sem=h1rn.at[c],
                device_id=prv, device_id_type=MESH,
            )
            rdp.start()
            h1_rdmas.extend((rdn, rdp))

        h2_rdmas = []
        for c in range(C):
            offm = foreign_base + rm1 * nq + c * sub
            rcv = pltpu.make_async_remote_copy(
                src_ref=out_ref.at[pl.ds(offm, sub), :],
                dst_ref=out_ref.at[pl.ds(offm, sub), :],
                send_sem=h1sp.at[c], recv_sem=h1rp.at[c],
                device_id=prv, device_id_type=MESH,
            )
            rcv.wait_recv()
            if c < h:
                snd = pltpu.make_async_remote_copy(
                    src_ref=out_ref.at[pl.ds(offm, sub), :],
                    dst_ref=out_ref.at[pl.ds(offm, sub), :],
                    send_sem=h2sn.at[c], recv_sem=h2rp.at[c],
                    device_id=nxt, device_id_type=MESH,
                )
                snd.start()
                h2_rdmas.append(snd)
            offp = foreign_base + rp1 * nq + c * sub
            rcv2 = pltpu.make_async_remote_copy(
                src_ref=out_ref.at[pl.ds(offp, sub), :],
                dst_ref=out_ref.at[pl.ds(offp, sub), :],
                send_sem=h1sn.at[c], recv_sem=h1rn.at[c],
                device_id=nxt, device_id_type=MESH,
            )
            rcv2.wait_recv()
            if c >= h:
                snd2 = pltpu.make_async_remote_copy(
                    src_ref=out_ref.at[pl.ds(offp, sub), :],
                    dst_ref=out_ref.at[pl.ds(offp, sub), :],
                    send_sem=h2sp.at[c - h], recv_sem=h2rn.at[c - h],
                    device_id=prv, device_id_type=MESH,
                )
                snd2.start()
                h2_rdmas.append(snd2)

        for c in range(C):
            off2 = foreign_base + rp2 * nq + c * sub
            sem = h2rp.at[c] if c < h else h2rn.at[c - h]
            rcv = pltpu.make_async_remote_copy(
                src_ref=out_ref.at[pl.ds(off2, sub), :],
                dst_ref=out_ref.at[pl.ds(off2, sub), :],
                send_sem=h2sn.at[c % h], recv_sem=sem,
                device_id=prv, device_id_type=MESH,
            )
            rcv.wait_recv()

        for rd in z_rdmas:
            rd.wait_send()
        for rd in h1_rdmas:
            rd.wait_send()
        for rd in h2_rdmas:
            rd.wait_send()
        louts[-2].wait()
        louts[-1].wait()

    out_shape = jax.ShapeDtypeStruct((2 * m_per, n), x.dtype)
    return pl.pallas_call(
        body,
        out_shape=out_shape,
        in_specs=[pl.BlockSpec(memory_space=pl.ANY)],
        out_specs=pl.BlockSpec(memory_space=pl.ANY),
        scratch_shapes=[
            pltpu.VMEM((2, lrows, n), x.dtype),
            pltpu.SemaphoreType.DMA((2,)),
            pltpu.SemaphoreType.DMA((2,)),
            pltpu.SemaphoreType.DMA((C,)),
            pltpu.SemaphoreType.DMA((C,)),
            pltpu.SemaphoreType.DMA((C,)),
            pltpu.SemaphoreType.DMA((C,)),
            pltpu.SemaphoreType.DMA((C,)),
            pltpu.SemaphoreType.DMA((C,)),
            pltpu.SemaphoreType.DMA((C // 2,)),
            pltpu.SemaphoreType.DMA((C // 2,)),
            pltpu.SemaphoreType.DMA((C // 2,)),
            pltpu.SemaphoreType.DMA((C // 2,)),
        ],
        compiler_params=pltpu.CompilerParams(collective_id=0),
    )(x)
